# Initial kernel scaffold; baseline (speedup 1.0000x reference)
import jax
import jax.numpy as jnp
from jax.experimental import pallas as pl


def kernel(x, energy_threshold):
    raise NotImplementedError("write your pallas kernel here")


# speedup vs baseline: 4.6503x; 4.6503x over previous
import functools

import numpy as np
import jax
import jax.numpy as jnp
from jax.experimental import pallas as pl
from jax.experimental.pallas import tpu as pltpu

B, S, D = 4, 4096, 1024
F = S // 2 + 1
FP = 2176


def _dft_consts(s, f, fp):
    k = np.arange(fp, dtype=np.float64)[:, None]
    n = np.arange(s, dtype=np.float64)[None, :]
    theta = 2.0 * np.pi * k * n / s
    valid = (k < f).astype(np.float64)
    c = np.cos(theta) * valid
    sn = np.sin(theta) * valid
    w = np.full((fp, 1), 2.0 / s)
    w[0, 0] = 1.0 / s
    if f - 1 < fp:
        w[f - 1, 0] = 1.0 / s
    w = w * valid
    bf = jnp.bfloat16
    return (c.astype(bf), sn.astype(bf),
            (w * c).astype(bf), (w * sn).astype(bf))


def _k1_body(c_ref, s_ref, x_ref, xc_ref, xs_ref, e_ref):
    xb = x_ref[0]
    acc_c = jnp.dot(c_ref[...], xb, preferred_element_type=jnp.float32)
    acc_s = jnp.dot(s_ref[...], xb, preferred_element_type=jnp.float32)
    xc_ref[0] = acc_c.astype(xc_ref.dtype)
    xs_ref[0] = acc_s.astype(xs_ref.dtype)
    e_ref[0] = jnp.sum(acc_c * acc_c + acc_s * acc_s, axis=1, keepdims=True)


def _k2_body(f, fp, chunk, ecol_ref, erow_ref, thr_ref, mask_ref, rank_ref):
    erow = erow_ref[...]
    colidx = jax.lax.broadcasted_iota(jnp.int32, (1, fp), 1)
    colvalid = colidx < f
    total = jnp.sum(erow)
    t = thr_ref[0, 0]
    thr_adj = 0.95 + 0.1 / (1.0 + jnp.exp(-t))
    thr_total = thr_adj * total

    nchunks = fp // chunk
    kcnt = jnp.zeros((), jnp.float32)
    for ci in range(nchunks):
        r0 = ci * chunk
        e_r = ecol_ref[r0:r0 + chunk, :]
        rowidx = jax.lax.broadcasted_iota(jnp.int32, (chunk, 1), 0) + r0
        gt = jnp.where((erow > e_r) & colvalid, 1.0, 0.0)
        eqb = jnp.where((erow == e_r) & (colidx < rowidx), 1.0, 0.0)
        rank_ref[r0:r0 + chunk, :] = (
            jnp.sum(gt, axis=1, keepdims=True)
            + jnp.sum(eqb, axis=1, keepdims=True))
        cum_r = jnp.sum(jnp.where(colidx <= rowidx, erow, 0.0),
                        axis=1, keepdims=True)
        below = (cum_r < thr_total) & (rowidx < f)
        kcnt = kcnt + jnp.sum(jnp.where(below, 1.0, 0.0))

    kval = jnp.minimum(kcnt + 1.0, jnp.float32(f))
    for ci in range(nchunks):
        r0 = ci * chunk
        rowidx = jax.lax.broadcasted_iota(jnp.int32, (chunk, 1), 0) + r0
        keep = (rank_ref[r0:r0 + chunk, :] < kval) & (rowidx < f)
        mask_ref[r0:r0 + chunk, :] = jnp.where(
            keep, 1.0, 0.0).astype(mask_ref.dtype)


def _k3_body(nki, ci_ref, si_ref, xc_ref, xs_ref, m_ref, x_ref, o_ref):
    ki = pl.program_id(2)

    @pl.when(ki == 0)
    def _():
        o_ref[0] = jnp.zeros_like(o_ref[0])

    m = m_ref[...].astype(jnp.bfloat16)
    xcm = xc_ref[0] * m
    xsm = xs_ref[0] * m
    dn = (((0,), (0,)), ((), ()))
    acc = jax.lax.dot_general(ci_ref[...], xcm, dn,
                              preferred_element_type=jnp.float32)
    acc = acc + jax.lax.dot_general(si_ref[...], xsm, dn,
                                    preferred_element_type=jnp.float32)
    o_ref[0] += acc

    @pl.when(ki == nki - 1)
    def _():
        o_ref[0] = o_ref[0] * x_ref[0]


def _build(b, s, d, f, fp, interpret=False):
    fk = fp // 8
    bn = s // 4
    nki = fp // fk
    c_bf, s_bf, ci_bf, si_bf = _dft_consts(s, f, fp)

    k1 = pl.pallas_call(
        _k1_body,
        grid=(b, fp // fk),
        in_specs=[
            pl.BlockSpec((fk, s), lambda bi, fi: (fi, 0)),
            pl.BlockSpec((fk, s), lambda bi, fi: (fi, 0)),
            pl.BlockSpec((1, s, d), lambda bi, fi: (bi, 0, 0)),
        ],
        out_specs=[
            pl.BlockSpec((1, fk, d), lambda bi, fi: (bi, fi, 0)),
            pl.BlockSpec((1, fk, d), lambda bi, fi: (bi, fi, 0)),
            pl.BlockSpec((1, fk, 1), lambda bi, fi: (bi, fi, 0)),
        ],
        out_shape=[
            jax.ShapeDtypeStruct((b, fp, d), jnp.bfloat16),
            jax.ShapeDtypeStruct((b, fp, d), jnp.bfloat16),
            jax.ShapeDtypeStruct((b, fp, 1), jnp.float32),
        ],
        interpret=interpret,
    )

    k2 = pl.pallas_call(
        functools.partial(_k2_body, f, fp, fk),
        in_specs=[
            pl.BlockSpec((fp, 1), lambda: (0, 0)),
            pl.BlockSpec((1, fp), lambda: (0, 0)),
            pl.BlockSpec(memory_space=pltpu.SMEM),
        ],
        out_specs=pl.BlockSpec((fp, 1), lambda: (0, 0)),
        out_shape=jax.ShapeDtypeStruct((fp, 1), jnp.bfloat16),
        scratch_shapes=[pltpu.VMEM((fp, 1), jnp.float32)],
        interpret=interpret,
    )

    k3 = pl.pallas_call(
        functools.partial(_k3_body, nki),
        grid=(b, s // bn, nki),
        in_specs=[
            pl.BlockSpec((fk, bn), lambda bi, ni, ki: (ki, ni)),
            pl.BlockSpec((fk, bn), lambda bi, ni, ki: (ki, ni)),
            pl.BlockSpec((1, fk, d), lambda bi, ni, ki: (bi, ki, 0)),
            pl.BlockSpec((1, fk, d), lambda bi, ni, ki: (bi, ki, 0)),
            pl.BlockSpec((fk, 1), lambda bi, ni, ki: (ki, 0)),
            pl.BlockSpec((1, bn, d), lambda bi, ni, ki: (bi, ni, 0)),
        ],
        out_specs=pl.BlockSpec((1, bn, d), lambda bi, ni, ki: (bi, ni, 0)),
        out_shape=jax.ShapeDtypeStruct((b, s, d), jnp.float32),
        interpret=interpret,
    )

    def run(x, energy_threshold):
        x_bf = x.astype(jnp.bfloat16)
        xc, xs, e_part = k1(c_bf, s_bf, x_bf)
        energy = jnp.sum(e_part, axis=0)
        thr = jnp.asarray(energy_threshold,
                          jnp.float32).reshape(1, 1)
        mask = k2(energy, energy.reshape(1, fp), thr)
        return k3(ci_bf, si_bf, xc, xs, mask, x)

    return run


def kernel(x, energy_threshold):
    return _build(B, S, D, F, FP)(x, energy_threshold)
